# Initial kernel scaffold; baseline (speedup 1.0000x reference)
import jax
import jax.numpy as jnp
from jax.experimental import pallas as pl


def kernel(x, W):
    raise NotImplementedError("write your pallas kernel here")


# speedup vs baseline: 6.1907x; 6.1907x over previous
import functools

import jax
import jax.numpy as jnp
import numpy as np
from jax import lax
from jax.experimental import pallas as pl
from jax.experimental.pallas import tpu as pltpu
from jax.experimental.pallas import tpu_sc as plsc

_FIELD_DIMS = [3846] * 25 + [3850]
_F = 26
_D = 16
_TOTAL = 100000
_OFFS = np.array((0, *np.cumsum(_FIELD_DIMS)[:-1]), dtype=np.int32)
_B = 4096
_PAIRS = [(i, j) for i in range(_F) for j in range(i + 1, _F)]
_NP = len(_PAIRS)

_IDX_PER_DMA = 128
_NDMA = 6
_GPS = _NDMA * _IDX_PER_DMA

_NW = 32
_SPW = _B // _NW

_mesh = plsc.VectorSubcoreMesh(core_axis_name="c", subcore_axis_name="s")


@functools.partial(
    pl.kernel,
    out_type=jax.ShapeDtypeStruct((_B, _NP, _D), jnp.float32),
    mesh=_mesh,
    compiler_params=pltpu.CompilerParams(use_tc_tiling_on_sc=False),
    scratch_types=[
        pltpu.VMEM((_NDMA, _IDX_PER_DMA), jnp.int32),
        pltpu.VMEM((_GPS, _D), jnp.float32),
        pltpu.VMEM((_NP, _D), jnp.float32),
        pltpu.SemaphoreType.DMA,
    ],
)
def _ffm_sc(idx_hbm, w_hbm, out_hbm, idx_v, rows_v, out_v, gsem):
    wid = lax.axis_index("s") * 2 + lax.axis_index("c")
    base = wid * _SPW

    @pl.loop(base, base + _SPW)
    def _body(s):
        pltpu.sync_copy(idx_hbm.at[s], idx_v)
        descs = [
            pltpu.async_copy(
                w_hbm.at[idx_v.at[q]],
                rows_v.at[pl.ds(q * _IDX_PER_DMA, _IDX_PER_DMA)],
                gsem,
            )
            for q in range(_NDMA)
        ]
        for d in descs:
            d.wait()
        for p, (i, j) in enumerate(_PAIRS):
            out_v[p] = rows_v[i * _F + j] * rows_v[j * _F + i]
        pltpu.sync_copy(out_v, out_hbm.at[s])


def kernel(x, W):
    gidx = x + jnp.asarray(_OFFS)[None, :]
    tab = (jnp.arange(_F, dtype=jnp.int32) * _TOTAL)[None, :, None]
    flat = tab + gidx[:, None, :]
    flat = flat.reshape(_B, _F * _F)
    flat = jnp.pad(flat, ((0, 0), (0, _GPS - _F * _F)))
    idx3 = flat.reshape(_B, _NDMA, _IDX_PER_DMA)
    w2 = W.reshape(_F * _TOTAL, _D)
    return _ffm_sc(idx3, w2)


# speedup vs baseline: 10.3588x; 1.6733x over previous
import functools

import jax
import jax.numpy as jnp
import numpy as np
from jax import lax
from jax.experimental import pallas as pl
from jax.experimental.pallas import tpu as pltpu
from jax.experimental.pallas import tpu_sc as plsc

_FIELD_DIMS = [3846] * 25 + [3850]
_F = 26
_D = 16
_TOTAL = 100000
_OFFS = np.array((0, *np.cumsum(_FIELD_DIMS)[:-1]), dtype=np.int32)
_B = 4096
_PAIRS = [(i, j) for i in range(_F) for j in range(i + 1, _F)]
_NP = len(_PAIRS)

_GPS = 680
_G = 4
_ROWS = _G * _GPS
_OUTR = _G * _NP

_NW = 32
_SPW = _B // _NW
_NGRP = _SPW // _G
_NGRP_T = _B // _G

_mesh = plsc.VectorSubcoreMesh(core_axis_name="c", subcore_axis_name="s")


@functools.partial(
    pl.kernel,
    out_type=jax.ShapeDtypeStruct((_NGRP_T, _OUTR, _D), jnp.float32),
    mesh=_mesh,
    compiler_params=pltpu.CompilerParams(use_tc_tiling_on_sc=False),
    scratch_types=[
        pltpu.VMEM((2, _ROWS), jnp.int32),
        pltpu.VMEM((2, _ROWS, _D), jnp.float32),
        pltpu.VMEM((_OUTR, _D), jnp.float32),
        pltpu.SemaphoreType.DMA,
        pltpu.SemaphoreType.DMA,
        pltpu.SemaphoreType.DMA,
        pltpu.SemaphoreType.DMA,
        pltpu.SemaphoreType.DMA,
    ],
)
def _ffm_sc(idx_hbm, w_hbm, out_hbm, idx_v, rows_v, out_v,
            isem0, isem1, gsem0, gsem1, osem):
    isem = (isem0, isem1)
    gsem = (gsem0, gsem1)
    wid = lax.axis_index("s") * 2 + lax.axis_index("c")
    gbase = wid * _NGRP

    def fetch_idx(g, slot):
        pltpu.async_copy(idx_hbm.at[gbase + g], idx_v.at[slot], isem[slot])

    def wait_idx(slot):
        pltpu.make_async_copy(idx_hbm.at[0], idx_v.at[slot], isem[slot]).wait()

    def fire_gathers(slot):
        for t in range(_G):
            sl = pl.ds(t * _GPS, _GPS)
            pltpu.async_copy(w_hbm.at[idx_v.at[slot, sl]],
                             rows_v.at[slot, sl], gsem[slot])

    def drain_gathers(slot):
        pltpu.make_async_copy(w_hbm.at[pl.ds(0, _ROWS)],
                              rows_v.at[slot], gsem[slot]).wait()

    def drain_out():
        pltpu.make_async_copy(w_hbm.at[pl.ds(0, _OUTR)], out_v, osem).wait()

    def compute(slot):
        rows = rows_v.at[slot]

        @pl.loop(0, _G)
        def _sample(t):
            rbase = t * _GPS
            obase = t * _NP
            for p, (i, j) in enumerate(_PAIRS):
                out_v[obase + p] = (rows[rbase + i * _F + j]
                                    * rows[rbase + j * _F + i])

    fetch_idx(0, 0)
    fetch_idx(1, 1)
    wait_idx(0)
    fire_gathers(0)

    @pl.loop(0, _NGRP, step=2)
    def _body(gg):
        for rr in range(2):
            g = gg + rr

            @pl.when(g + 1 < _NGRP)
            def _():
                wait_idx(1 - rr)
                fire_gathers(1 - rr)

            drain_gathers(rr)

            @pl.when(g + 2 < _NGRP)
            def _():
                fetch_idx(g + 2, rr)

            if rr == 0:
                @pl.when(g > 0)
                def _():
                    drain_out()
            else:
                drain_out()
            compute(rr)
            pltpu.async_copy(out_v, out_hbm.at[gbase + g], osem)

    drain_out()


def kernel(x, W):
    gidx = x + jnp.asarray(_OFFS)[None, :]
    tab = (jnp.arange(_F, dtype=jnp.int32) * _TOTAL)[None, :, None]
    flat = tab + gidx[:, None, :]
    flat = flat.reshape(_B, _F * _F)
    flat = jnp.pad(flat, ((0, 0), (0, _GPS - _F * _F)))
    idx3 = flat.reshape(_NGRP_T, _ROWS)
    w2 = W.reshape(_F * _TOTAL, _D)
    out = _ffm_sc(idx3, w2)
    return out.reshape(_B, _NP, _D)


# speedup vs baseline: 10.8918x; 1.0514x over previous
import functools

import jax
import jax.numpy as jnp
import numpy as np
from jax import lax
from jax.experimental import pallas as pl
from jax.experimental.pallas import tpu as pltpu
from jax.experimental.pallas import tpu_sc as plsc

_FIELD_DIMS = [3846] * 25 + [3850]
_F = 26
_D = 16
_TOTAL = 100000
_OFFS = np.array((0, *np.cumsum(_FIELD_DIMS)[:-1]), dtype=np.int32)
_B = 4096
_PAIRS = [(i, j) for i in range(_F) for j in range(i + 1, _F)]
_NP = len(_PAIRS)

_G = 4
_IPG = _G * _F
_OUTR = _G * _NP

_NW = 32
_SPW = _B // _NW
_NGRP = _SPW // _G
_NGRP_T = _B // _G

_mesh = plsc.VectorSubcoreMesh(core_axis_name="c", subcore_axis_name="s")


@functools.partial(
    pl.kernel,
    out_type=jax.ShapeDtypeStruct((_B, _NP, _D), jnp.float32),
    mesh=_mesh,
    compiler_params=pltpu.CompilerParams(use_tc_tiling_on_sc=False),
    scratch_types=[
        pltpu.VMEM((2, _IPG), jnp.int32),
        pltpu.VMEM((2, _F * _IPG, _D), jnp.float32),
        pltpu.VMEM((_OUTR, _D), jnp.float32),
        pltpu.SemaphoreType.DMA,
        pltpu.SemaphoreType.DMA,
        pltpu.SemaphoreType.DMA,
        pltpu.SemaphoreType.DMA,
        pltpu.SemaphoreType.DMA,
    ],
)
def _ffm_sc(idx_hbm, w_hbm, out_hbm, idx_v, rows_v, out_v,
            isem0, isem1, gsem0, gsem1, osem):
    isem = (isem0, isem1)
    gsem = (gsem0, gsem1)
    wid = lax.axis_index("s") * 2 + lax.axis_index("c")
    gbase = wid * _NGRP

    def fetch_idx(g, slot):
        pltpu.async_copy(idx_hbm.at[gbase + g], idx_v.at[slot], isem[slot])

    def wait_idx(slot):
        pltpu.make_async_copy(idx_hbm.at[0], idx_v.at[slot], isem[slot]).wait()

    def fire_gathers(slot):
        for i in range(_F):
            pltpu.async_copy(w_hbm.at[i].at[idx_v.at[slot]],
                             rows_v.at[slot, pl.ds(i * _IPG, _IPG)],
                             gsem[slot])

    def drain_gathers(slot):
        pltpu.make_async_copy(w_hbm.at[0].at[pl.ds(0, _F * _IPG)],
                              rows_v.at[slot], gsem[slot]).wait()

    def drain_out():
        pltpu.make_async_copy(w_hbm.at[0].at[pl.ds(0, _OUTR)],
                              out_v, osem).wait()

    def compute(slot):
        rows = rows_v.at[slot]

        @pl.loop(0, _G)
        def _sample(t):
            rbase = t * _F
            obase = t * _NP
            for p, (i, j) in enumerate(_PAIRS):
                out_v[obase + p] = (rows[i * _IPG + rbase + j]
                                    * rows[j * _IPG + rbase + i])

    fetch_idx(0, 0)
    fetch_idx(1, 1)
    wait_idx(0)
    fire_gathers(0)

    @pl.loop(0, _NGRP, step=2)
    def _body(gg):
        for rr in range(2):
            g = gg + rr

            @pl.when(g + 1 < _NGRP)
            def _():
                wait_idx(1 - rr)
                fire_gathers(1 - rr)

            drain_gathers(rr)

            @pl.when(g + 2 < _NGRP)
            def _():
                fetch_idx(g + 2, rr)

            if rr == 0:
                @pl.when(g > 0)
                def _():
                    drain_out()
            else:
                drain_out()
            compute(rr)
            s0 = (gbase + g) * _G
            for t in range(_G):
                pltpu.async_copy(out_v.at[pl.ds(t * _NP, _NP)],
                                 out_hbm.at[s0 + t], osem)

    drain_out()


def kernel(x, W):
    gidx = x + jnp.asarray(_OFFS)[None, :]
    idx2 = gidx.reshape(_NGRP_T, _IPG)
    return _ffm_sc(idx2, W)


# speedup vs baseline: 38.3658x; 3.5225x over previous
import functools

import jax
import jax.numpy as jnp
import numpy as np
from jax import lax
from jax.experimental import pallas as pl
from jax.experimental.pallas import tpu as pltpu
from jax.experimental.pallas import tpu_sc as plsc

_FIELD_DIMS = [3846] * 25 + [3850]
_F = 26
_D = 16
_V = 100000
_OFFS = np.array((0, *np.cumsum(_FIELD_DIMS)[:-1]), dtype=np.int32)
_B = 4096
_PAIRS = [(i, j) for i in range(_F) for j in range(i + 1, _F)]
_NP = len(_PAIRS)

_SLABW = 3856
_Q = 4
_NQ = _D // _Q
_NG = _B // 16

_NW = 32

_BASE = [i * (2 * _F - 1 - i) // 2 for i in range(_F)]
_ADJ = (_OFFS % 8).astype(np.int32)

_mesh = plsc.VectorSubcoreMesh(core_axis_name="c", subcore_axis_name="s")


@functools.partial(
    pl.kernel,
    out_type=jax.ShapeDtypeStruct((_NP, _D, _B), jnp.float32),
    mesh=_mesh,
    compiler_params=pltpu.CompilerParams(use_tc_tiling_on_sc=False,
                                         needs_layout_passes=False),
    scratch_types=[
        pltpu.VMEM((2 * _B,), jnp.int32),
        pltpu.VMEM((2 * _B,), jnp.int32),
        pltpu.VMEM((2, _Q, _SLABW), jnp.float32),
        pltpu.VMEM((2, _Q, _SLABW), jnp.float32),
        pltpu.VMEM((2, _Q, _B), jnp.float32),
        pltpu.SemaphoreType.DMA,
        pltpu.SemaphoreType.DMA,
        pltpu.SemaphoreType.DMA,
        pltpu.SemaphoreType.DMA,
        pltpu.SemaphoreType.DMA,
        pltpu.SemaphoreType.DMA,
    ],
)
def _ffm_sc(xadj_hbm, wt_hbm, ot_hbm, ia_v, ib_v,
            sa_v, sb_v, out_v, isem0, isem1, ssem0, ssem1, osem0, osem1):
    isem = (isem0, isem1)
    ssem = (ssem0, ssem1)
    osem = (osem0, osem1)
    wid = lax.axis_index("s") * 2 + lax.axis_index("c")
    p_lo = wid * _NP // _NW
    p_hi = (wid + 1) * _NP // _NW

    def params(k):
        i = jnp.int32(0)
        for t in range(1, _F):
            i = i + jnp.where(k >= _BASE[t], 1, 0).astype(jnp.int32)
        j = k - i * (2 * _F - 1 - i) // 2 + i + 1
        sa = pl.multiple_of(jnp.bitwise_and(3846 * j, -8), 8)
        sb = pl.multiple_of(jnp.bitwise_and(3846 * i, -8), 8)
        return i, j, sa, j, i, sb

    def fetch_idx(fa, fb, kk):
        sl = pl.ds(kk * _B, _B)
        pltpu.async_copy(xadj_hbm.at[fa], ia_v.at[sl], isem[kk])
        pltpu.async_copy(xadj_hbm.at[fb], ib_v.at[sl], isem[kk])

    def wait_idx(kk):
        sl = pl.ds(kk * _B, _B)
        pltpu.make_async_copy(xadj_hbm.at[0], ia_v.at[sl], isem[kk]).wait()
        pltpu.make_async_copy(xadj_hbm.at[0], ib_v.at[sl], isem[kk]).wait()

    def fire_slabs(pa, sa, pb, sb, q, slot):
        rows = pl.ds(q * _Q, _Q)
        pltpu.async_copy(wt_hbm.at[pa, rows, pl.ds(sa, _SLABW)],
                         sa_v.at[slot], ssem[slot])
        pltpu.async_copy(wt_hbm.at[pb, rows, pl.ds(sb, _SLABW)],
                         sb_v.at[slot], ssem[slot])

    def drain_slabs(slot):
        dummy = wt_hbm.at[0, pl.ds(0, _Q), pl.ds(0, _SLABW)]
        pltpu.make_async_copy(dummy, sa_v.at[slot], ssem[slot]).wait()
        pltpu.make_async_copy(dummy, sb_v.at[slot], ssem[slot]).wait()

    def drain_out(slot):
        dummy = ot_hbm.at[0, pl.ds(0, _Q), :]
        pltpu.make_async_copy(dummy, out_v.at[slot], osem[slot]).wait()

    def pair_body(k, kk):
        wait_idx(kk)

        nxt = jnp.minimum(k + 1, _NP - 1)
        npa, nfa, nsa, npb, nfb, nsb = params(nxt)

        @pl.when(k + 1 < p_hi)
        def _():
            fetch_idx(nfa, nfb, 1 - kk)

        pa, fa, sa, pb, fb, sb = params(k)
        del fa, fb

        for q in range(_NQ):
            drain_slabs(q % 2)
            if q < _NQ - 1:
                fire_slabs(pa, sa, pb, sb, q + 1, (q + 1) % 2)
            else:
                @pl.when(k + 1 < p_hi)
                def _():
                    fire_slabs(npa, nsa, npb, nsb, 0, 0)

            @pl.when((k - p_lo) * _NQ + q >= 2)
            def _():
                drain_out(q % 2)

            @pl.loop(0, _NG, unroll=4)
            def _grp(g):
                iva = ia_v[pl.ds(kk * _B + g * 16, 16)]
                ivb = ib_v[pl.ds(kk * _B + g * 16, 16)]
                for d in range(_Q):
                    a = plsc.load_gather(sa_v.at[q % 2, d], [iva])
                    b = plsc.load_gather(sb_v.at[q % 2, d], [ivb])
                    out_v[q % 2, d, pl.ds(g * 16, 16)] = a * b

            pltpu.async_copy(out_v.at[q % 2],
                             ot_hbm.at[k, pl.ds(q * _Q, _Q), :],
                             osem[q % 2])

    pa0, fa0, sa0, pb0, fb0, sb0 = params(p_lo)
    fetch_idx(fa0, fb0, 0)
    fire_slabs(pa0, sa0, pb0, sb0, 0, 0)

    @pl.loop(p_lo, p_hi, step=2)
    def _pairs(gg):
        pair_body(gg, 0)

        @pl.when(gg + 1 < p_hi)
        def _():
            pair_body(gg + 1, 1)

    drain_out(0)
    drain_out(1)


def kernel(x, W):
    wt = jnp.transpose(W, (0, 2, 1))
    xadj = x.T + jnp.asarray(_ADJ)[:, None]
    ot = _ffm_sc(xadj, wt)
    return jnp.transpose(ot, (2, 0, 1))


# speedup vs baseline: 56.4235x; 1.4707x over previous
import functools

import jax
import jax.numpy as jnp
import numpy as np
from jax import lax
from jax.experimental import pallas as pl
from jax.experimental.pallas import tpu as pltpu
from jax.experimental.pallas import tpu_sc as plsc

_FIELD_DIMS = [3846] * 25 + [3850]
_F = 26
_D = 16
_V = 100000
_OFFS = np.array((0, *np.cumsum(_FIELD_DIMS)[:-1]), dtype=np.int32)
_B = 4096
_PAIRS = [(i, j) for i in range(_F) for j in range(i + 1, _F)]
_NP = len(_PAIRS)

_SLABW = 3856
_Q = 4
_NQ = _D // _Q
_NG = _B // 16

_NW = 32

_BASE = [i * (2 * _F - 1 - i) // 2 for i in range(_F)]
_ADJ = (_OFFS % 8).astype(np.int32)

_mesh = plsc.VectorSubcoreMesh(core_axis_name="c", subcore_axis_name="s")


@functools.partial(
    pl.kernel,
    out_type=jax.ShapeDtypeStruct((_NP, _D, _B), jnp.float32),
    mesh=_mesh,
    compiler_params=pltpu.CompilerParams(use_tc_tiling_on_sc=False,
                                         needs_layout_passes=False),
    scratch_types=[
        pltpu.VMEM((2 * _B,), jnp.int32),
        pltpu.VMEM((2 * _B,), jnp.int32),
        pltpu.VMEM((2, _Q, _SLABW), jnp.float32),
        pltpu.VMEM((2, _Q, _SLABW), jnp.float32),
        pltpu.VMEM((2, _Q, _B), jnp.float32),
        pltpu.SemaphoreType.DMA,
        pltpu.SemaphoreType.DMA,
        pltpu.SemaphoreType.DMA,
        pltpu.SemaphoreType.DMA,
        pltpu.SemaphoreType.DMA,
        pltpu.SemaphoreType.DMA,
    ],
)
def _ffm_sc(xadj_hbm, wt_hbm, ot_hbm, ia_v, ib_v,
            sa_v, sb_v, out_v, isem0, isem1, ssem0, ssem1, osem0, osem1):
    isem = (isem0, isem1)
    ssem = (ssem0, ssem1)
    osem = (osem0, osem1)
    wid = lax.axis_index("s") * 2 + lax.axis_index("c")
    p_lo = wid * _NP // _NW
    p_hi = (wid + 1) * _NP // _NW

    def params(k):
        i = jnp.int32(0)
        for t in range(1, _F):
            i = i + jnp.where(k >= _BASE[t], 1, 0).astype(jnp.int32)
        j = k - i * (2 * _F - 1 - i) // 2 + i + 1
        sa = pl.multiple_of(jnp.bitwise_and(3846 * j, -8), 8)
        sb = pl.multiple_of(jnp.bitwise_and(3846 * i, -8), 8)
        return i, j, sa, j, i, sb

    def fetch_idx(fa, fb, kk):
        sl = pl.ds(kk * _B, _B)
        pltpu.async_copy(xadj_hbm.at[fa], ia_v.at[sl], isem[kk])
        pltpu.async_copy(xadj_hbm.at[fb], ib_v.at[sl], isem[kk])

    def wait_idx(kk):
        sl = pl.ds(kk * _B, _B)
        pltpu.make_async_copy(xadj_hbm.at[0], ia_v.at[sl], isem[kk]).wait()
        pltpu.make_async_copy(xadj_hbm.at[0], ib_v.at[sl], isem[kk]).wait()

    def fire_slabs(pa, sa, pb, sb, q, slot):
        rows = pl.ds(q * _Q, _Q)
        pltpu.async_copy(wt_hbm.at[pa, rows, pl.ds(sa, _SLABW)],
                         sa_v.at[slot], ssem[slot])
        pltpu.async_copy(wt_hbm.at[pb, rows, pl.ds(sb, _SLABW)],
                         sb_v.at[slot], ssem[slot])

    def drain_slabs(slot):
        dummy = wt_hbm.at[0, pl.ds(0, _Q), pl.ds(0, _SLABW)]
        pltpu.make_async_copy(dummy, sa_v.at[slot], ssem[slot]).wait()
        pltpu.make_async_copy(dummy, sb_v.at[slot], ssem[slot]).wait()

    def drain_out(slot):
        dummy = ot_hbm.at[0, pl.ds(0, _Q), :]
        pltpu.make_async_copy(dummy, out_v.at[slot], osem[slot]).wait()

    def pair_body(k, kk):
        wait_idx(kk)

        nxt = jnp.minimum(k + 1, _NP - 1)
        npa, nfa, nsa, npb, nfb, nsb = params(nxt)

        @pl.when(k + 1 < p_hi)
        def _():
            fetch_idx(nfa, nfb, 1 - kk)

        pa, fa, sa, pb, fb, sb = params(k)
        del fa, fb

        for q in range(_NQ):
            drain_slabs(q % 2)
            if q < _NQ - 1:
                fire_slabs(pa, sa, pb, sb, q + 1, (q + 1) % 2)
            else:
                @pl.when(k + 1 < p_hi)
                def _():
                    fire_slabs(npa, nsa, npb, nsb, 0, 0)

            @pl.when((k - p_lo) * _NQ + q >= 2)
            def _():
                drain_out(q % 2)

            @plsc.parallel_loop(0, _NG, unroll=4)
            def _grp(g):
                iva = ia_v[pl.ds(kk * _B + g * 16, 16)]
                ivb = ib_v[pl.ds(kk * _B + g * 16, 16)]
                for d in range(_Q):
                    a = plsc.load_gather(sa_v.at[q % 2, d], [iva])
                    b = plsc.load_gather(sb_v.at[q % 2, d], [ivb])
                    out_v[q % 2, d, pl.ds(g * 16, 16)] = a * b

            pltpu.async_copy(out_v.at[q % 2],
                             ot_hbm.at[k, pl.ds(q * _Q, _Q), :],
                             osem[q % 2])

    pa0, fa0, sa0, pb0, fb0, sb0 = params(p_lo)
    fetch_idx(fa0, fb0, 0)
    fire_slabs(pa0, sa0, pb0, sb0, 0, 0)

    @pl.loop(p_lo, p_hi, step=2)
    def _pairs(gg):
        pair_body(gg, 0)

        @pl.when(gg + 1 < p_hi)
        def _():
            pair_body(gg + 1, 1)

    drain_out(0)
    drain_out(1)


def kernel(x, W):
    wt = jnp.transpose(W, (0, 2, 1))
    xadj = x.T + jnp.asarray(_ADJ)[:, None]
    ot = _ffm_sc(xadj, wt)
    return jnp.transpose(ot, (2, 0, 1))


# speedup vs baseline: 70.0306x; 1.2412x over previous
import functools

import jax
import jax.numpy as jnp
import numpy as np
from jax import lax
from jax.experimental import pallas as pl
from jax.experimental.pallas import tpu as pltpu
from jax.experimental.pallas import tpu_sc as plsc

_FIELD_DIMS = [3846] * 25 + [3850]
_F = 26
_D = 16
_V = 100000
_OFFS = np.array((0, *np.cumsum(_FIELD_DIMS)[:-1]), dtype=np.int32)
_B = 4096
_PAIRS = [(i, j) for i in range(_F) for j in range(i + 1, _F)]
_NP = len(_PAIRS)

_SLABW = 3856
_Q = 4
_NQ = _D // _Q
_NG = _B // 16

_NW = 32

_BASE = [i * (2 * _F - 1 - i) // 2 for i in range(_F)]
_ADJ = (_OFFS % 8).astype(np.int32)

_mesh = plsc.VectorSubcoreMesh(core_axis_name="c", subcore_axis_name="s")


@functools.partial(
    pl.kernel,
    out_type=jax.ShapeDtypeStruct((_NP, 2, _B // 128, 8, 128), jnp.float32),
    mesh=_mesh,
    compiler_params=pltpu.CompilerParams(use_tc_tiling_on_sc=False,
                                         needs_layout_passes=False),
    scratch_types=[
        pltpu.VMEM((2 * _B,), jnp.int32),
        pltpu.VMEM((2 * _B,), jnp.int32),
        pltpu.VMEM((2, _Q, _SLABW), jnp.float32),
        pltpu.VMEM((2, _Q, _SLABW), jnp.float32),
        pltpu.VMEM((2, _B // 128, _Q, 128), jnp.float32),
        pltpu.SemaphoreType.DMA,
        pltpu.SemaphoreType.DMA,
        pltpu.SemaphoreType.DMA,
        pltpu.SemaphoreType.DMA,
        pltpu.SemaphoreType.DMA,
        pltpu.SemaphoreType.DMA,
    ],
)
def _ffm_sc(xadj_hbm, wt_hbm, ot_hbm, ia_v, ib_v,
            sa_v, sb_v, out_v, isem0, isem1, ssem0, ssem1, osem0, osem1):
    isem = (isem0, isem1)
    ssem = (ssem0, ssem1)
    osem = (osem0, osem1)
    wid = lax.axis_index("s") * 2 + lax.axis_index("c")
    p_lo = wid * _NP // _NW
    p_hi = (wid + 1) * _NP // _NW

    def params(k):
        i = jnp.int32(0)
        for t in range(1, _F):
            i = i + jnp.where(k >= _BASE[t], 1, 0).astype(jnp.int32)
        j = k - i * (2 * _F - 1 - i) // 2 + i + 1
        sa = pl.multiple_of(jnp.bitwise_and(3846 * j, -8), 8)
        sb = pl.multiple_of(jnp.bitwise_and(3846 * i, -8), 8)
        return i, j, sa, j, i, sb

    def fetch_idx(fa, fb, kk):
        sl = pl.ds(kk * _B, _B)
        pltpu.async_copy(xadj_hbm.at[fa], ia_v.at[sl], isem[kk])
        pltpu.async_copy(xadj_hbm.at[fb], ib_v.at[sl], isem[kk])

    def wait_idx(kk):
        sl = pl.ds(kk * _B, _B)
        pltpu.make_async_copy(xadj_hbm.at[0], ia_v.at[sl], isem[kk]).wait()
        pltpu.make_async_copy(xadj_hbm.at[0], ib_v.at[sl], isem[kk]).wait()

    def fire_slabs(pa, sa, pb, sb, q, slot):
        rows = pl.ds(q * _Q, _Q)
        pltpu.async_copy(wt_hbm.at[pa, rows, pl.ds(sa, _SLABW)],
                         sa_v.at[slot], ssem[slot])
        pltpu.async_copy(wt_hbm.at[pb, rows, pl.ds(sb, _SLABW)],
                         sb_v.at[slot], ssem[slot])

    def drain_slabs(slot):
        dummy = wt_hbm.at[0, pl.ds(0, _Q), pl.ds(0, _SLABW)]
        pltpu.make_async_copy(dummy, sa_v.at[slot], ssem[slot]).wait()
        pltpu.make_async_copy(dummy, sb_v.at[slot], ssem[slot]).wait()

    def drain_out(slot):
        dummy = ot_hbm.at[0, 0, :, pl.ds(0, _Q), :]
        pltpu.make_async_copy(dummy, out_v.at[slot], osem[slot]).wait()

    def pair_body(k, kk):
        wait_idx(kk)

        nxt = jnp.minimum(k + 1, _NP - 1)
        npa, nfa, nsa, npb, nfb, nsb = params(nxt)

        @pl.when(k + 1 < p_hi)
        def _():
            fetch_idx(nfa, nfb, 1 - kk)

        pa, fa, sa, pb, fb, sb = params(k)
        del fa, fb

        for q in range(_NQ):
            drain_slabs(q % 2)
            if q < _NQ - 1:
                fire_slabs(pa, sa, pb, sb, q + 1, (q + 1) % 2)
            else:
                @pl.when(k + 1 < p_hi)
                def _():
                    fire_slabs(npa, nsa, npb, nsb, 0, 0)

            @pl.when((k - p_lo) * _NQ + q >= 2)
            def _():
                drain_out(q % 2)

            @plsc.parallel_loop(0, _NG, unroll=4)
            def _grp(g):
                iva = ia_v[pl.ds(kk * _B + g * 16, 16)]
                ivb = ib_v[pl.ds(kk * _B + g * 16, 16)]
                b1 = g // 8
                b0 = (g % 8) * 16
                for d in range(_Q):
                    a = plsc.load_gather(sa_v.at[q % 2, d], [iva])
                    b = plsc.load_gather(sb_v.at[q % 2, d], [ivb])
                    out_v[q % 2, b1, d, pl.ds(b0, 16)] = a * b

            pltpu.async_copy(out_v.at[q % 2],
                             ot_hbm.at[k, q // 2, :, pl.ds((q % 2) * _Q, _Q), :],
                             osem[q % 2])

    pa0, fa0, sa0, pb0, fb0, sb0 = params(p_lo)
    fetch_idx(fa0, fb0, 0)
    fire_slabs(pa0, sa0, pb0, sb0, 0, 0)

    @pl.loop(p_lo, p_hi, step=2)
    def _pairs(gg):
        pair_body(gg, 0)

        @pl.when(gg + 1 < p_hi)
        def _():
            pair_body(gg + 1, 1)

    drain_out(0)
    drain_out(1)


def kernel(x, W):
    wt = jnp.transpose(W, (0, 2, 1))
    xadj = x.T + jnp.asarray(_ADJ)[:, None]
    ot5 = _ffm_sc(xadj, wt)
    return jnp.transpose(ot5, (2, 4, 0, 1, 3)).reshape(_B, _NP, _D)
